# Initial kernel scaffold; baseline (speedup 1.0000x reference)
import functools

import jax
import jax.numpy as jnp
from jax import lax
from jax.experimental import pallas as pl
from jax.experimental.pallas import tpu as pltpu
from jax.experimental.pallas import tpu_sc as plsc

N_TOKENS = 16384
N_EXPERTS = 64
TOP_K = 8
L = 16
NC, NS = 2, 16
NW = NC * NS
TPW = N_TOKENS // NW
PAIRS = TPW // 2


def _make_kernel():
  mesh = plsc.VectorSubcoreMesh(core_axis_name="c", subcore_axis_name="s")

  @functools.partial(
      pl.kernel,
      out_type=[
          jax.ShapeDtypeStruct((N_TOKENS * TOP_K,), jnp.int32),
          jax.ShapeDtypeStruct((N_TOKENS * TOP_K,), jnp.float32),
      ],
      mesh=mesh,
      scratch_types=[
          pltpu.VMEM((TPW, N_EXPERTS), jnp.float32),
          pltpu.VMEM((TPW * TOP_K,), jnp.int32),
          pltpu.VMEM((TPW * TOP_K,), jnp.float32),
      ],
  )
  def _router_topk(logits_hbm, idx_hbm, vals_hbm, in_v, idx_v, vals_v):
    wid = lax.axis_index("s") * NC + lax.axis_index("c")
    row0 = wid * TPW
    pltpu.sync_copy(logits_hbm.at[pl.ds(row0, TPW)], in_v)

    iota = lax.iota(jnp.int32, L)
    m8 = iota < TOP_K
    shift8 = (iota + TOP_K) & (L - 1)
    idx_dn = jnp.minimum(iota + 1, L - 1)
    idx_up = jnp.maximum(iota - 1, 0)
    expert_ids = [iota + L * j for j in range(N_EXPERTS // L)]

    def gath(x, idx):
      return jnp.take_along_axis(x, idx, axis=0, mode="promise_in_bounds")

    def tie_fix(k, v):
      k_dn, v_dn = gath(k, idx_dn), gath(v, idx_dn)
      k_up, v_up = gath(k, idx_up), gath(v, idx_up)
      return jnp.where(
          k == k_dn, jnp.minimum(v, v_dn),
          jnp.where(k == k_up, jnp.maximum(v, v_up), v))

    def sort_fix(k, v):
      sk, sv = plsc.sort_key_val(k, v, descending=True)
      return sk, tie_fix(sk, sv)

    def merge(ak, av, bk, bv):
      mk = jnp.where(m8, ak, lax.rev(bk, (0,)))
      mv = jnp.where(m8, av, lax.rev(bv, (0,)))
      return sort_fix(mk, mv)

    def token_topk(t):
      x = [in_v[t, pl.ds(L * j, L)] for j in range(N_EXPERTS // L)]
      s = [sort_fix(x[j], expert_ids[j]) for j in range(N_EXPERTS // L)]
      ek, ev = merge(*s[0], *s[1])
      fk, fv = merge(*s[2], *s[3])
      gk, gv = merge(ek, ev, fk, fv)
      mx = jnp.max(jnp.maximum(jnp.maximum(x[0], x[1]),
                               jnp.maximum(x[2], x[3])))
      den = jnp.sum(jnp.exp(x[0] - mx) + jnp.exp(x[1] - mx)
                    + jnp.exp(x[2] - mx) + jnp.exp(x[3] - mx))
      return gv, jnp.exp(gk - mx) / den

    def pair_body(p, carry):
      i0, v0 = token_topk(2 * p)
      i1, v1 = token_topk(2 * p + 1)
      oi = jnp.where(m8, i0, gath(i1, shift8))
      ov = jnp.where(m8, v0, gath(v1, shift8))
      idx_v[pl.ds(p * L, L)] = oi
      vals_v[pl.ds(p * L, L)] = ov
      return carry

    lax.fori_loop(0, PAIRS, pair_body, 0)

    out0 = row0 * TOP_K
    pltpu.sync_copy(idx_v, idx_hbm.at[pl.ds(out0, TPW * TOP_K)])
    pltpu.sync_copy(vals_v, vals_hbm.at[pl.ds(out0, TPW * TOP_K)])

  return _router_topk


_ROUTER_TOPK = _make_kernel()


def kernel(router_logits):
  idx_flat, vals_flat = _ROUTER_TOPK(router_logits)
  return (idx_flat.reshape(N_TOKENS, TOP_K),
          vals_flat.reshape(N_TOKENS, TOP_K))


# speedup vs baseline: 1.0043x; 1.0043x over previous
import functools

import jax
import jax.numpy as jnp
from jax import lax
from jax.experimental import pallas as pl
from jax.experimental.pallas import tpu as pltpu
from jax.experimental.pallas import tpu_sc as plsc

N_TOKENS = 16384
N_EXPERTS = 64
TOP_K = 8
L = 16
NC, NS = 2, 16
NW = NC * NS
TPW = N_TOKENS // NW
PAIRS = TPW // 2


def _make_kernel():
  mesh = plsc.VectorSubcoreMesh(core_axis_name="c", subcore_axis_name="s")

  @functools.partial(
      pl.kernel,
      out_type=[
          jax.ShapeDtypeStruct((N_TOKENS * TOP_K,), jnp.int32),
          jax.ShapeDtypeStruct((N_TOKENS * TOP_K,), jnp.float32),
      ],
      mesh=mesh,
      compiler_params=pltpu.CompilerParams(needs_layout_passes=False),
      scratch_types=[
          pltpu.VMEM((TPW, N_EXPERTS), jnp.float32),
          pltpu.VMEM((TPW * TOP_K,), jnp.int32),
          pltpu.VMEM((TPW * TOP_K,), jnp.float32),
      ],
  )
  def _router_topk(logits_hbm, idx_hbm, vals_hbm, in_v, idx_v, vals_v):
    wid = lax.axis_index("s") * NC + lax.axis_index("c")
    row0 = wid * TPW
    pltpu.sync_copy(logits_hbm.at[pl.ds(row0, TPW)], in_v)

    iota = lax.iota(jnp.int32, L)
    m8 = iota < TOP_K
    shift8 = (iota + TOP_K) & (L - 1)
    idx_dn = jnp.minimum(iota + 1, L - 1)
    idx_up = jnp.maximum(iota - 1, 0)
    expert_ids = [iota + L * j for j in range(N_EXPERTS // L)]

    def gath(x, idx):
      return jnp.take_along_axis(x, idx, axis=0, mode="promise_in_bounds")

    def tie_fix(k, v):
      k_dn, v_dn = gath(k, idx_dn), gath(v, idx_dn)
      k_up, v_up = gath(k, idx_up), gath(v, idx_up)
      return jnp.where(
          k == k_dn, jnp.minimum(v, v_dn),
          jnp.where(k == k_up, jnp.maximum(v, v_up), v))

    def sort_fix(k, v):
      sk, sv = plsc.sort_key_val(k, v, descending=True)
      return sk, tie_fix(sk, sv)

    def merge(ak, av, bk, bv):
      mk = jnp.where(m8, ak, lax.rev(bk, (0,)))
      mv = jnp.where(m8, av, lax.rev(bv, (0,)))
      return sort_fix(mk, mv)

    def token_topk(t):
      x = [in_v[t, pl.ds(L * j, L)] for j in range(N_EXPERTS // L)]
      s = [sort_fix(x[j], expert_ids[j]) for j in range(N_EXPERTS // L)]
      ek, ev = merge(*s[0], *s[1])
      fk, fv = merge(*s[2], *s[3])
      gk, gv = merge(ek, ev, fk, fv)
      mx = jnp.max(jnp.maximum(jnp.maximum(x[0], x[1]),
                               jnp.maximum(x[2], x[3])))
      den = jnp.sum(jnp.exp(x[0] - mx) + jnp.exp(x[1] - mx)
                    + jnp.exp(x[2] - mx) + jnp.exp(x[3] - mx))
      return gv, jnp.exp(gk - mx) / den

    def pair_body(p, carry):
      i0, v0 = token_topk(2 * p)
      i1, v1 = token_topk(2 * p + 1)
      oi = jnp.where(m8, i0, gath(i1, shift8))
      ov = jnp.where(m8, v0, gath(v1, shift8))
      idx_v[pl.ds(p * L, L)] = oi
      vals_v[pl.ds(p * L, L)] = ov
      return carry

    lax.fori_loop(0, PAIRS, pair_body, 0)

    out0 = row0 * TOP_K
    pltpu.sync_copy(idx_v, idx_hbm.at[pl.ds(out0, TPW * TOP_K)])
    pltpu.sync_copy(vals_v, vals_hbm.at[pl.ds(out0, TPW * TOP_K)])

  return _router_topk


_ROUTER_TOPK = _make_kernel()


def kernel(router_logits):
  idx_flat, vals_flat = _ROUTER_TOPK(router_logits)
  return (idx_flat.reshape(N_TOKENS, TOP_K),
          vals_flat.reshape(N_TOKENS, TOP_K))
